# baseline (speedup 1.0000x reference)
import jax
import jax.numpy as jnp
from jax import lax
from jax.experimental import pallas as pl
from jax.experimental.pallas import tpu as pltpu
from jax.experimental.pallas import tpu_sc as plsc

B = 16384
N_CAT = 26
VOCAB = 1000
EMB = 77
N_COLS = 41
LANES = 128
NC = 2
NS = 16
NW = NC * NS
ROWS = B // NW
ROWS_H = ROWS // 2
IDX_H = ROWS_H * 32
CTAB = N_CAT * VOCAB
CTMAX = CTAB - 1
CT_PER = 1664
CT_VEC = CT_PER // 16


def _body(inp_hbm, tbl_hbm, out_hbm, inp_v, out_v, idx_v, gath_v,
          ctidx_v, ctg_v, ctab_s, sem, sem2):
    sid = lax.axis_index("s")
    wid = sid * NC + lax.axis_index("c")
    row0 = wid * ROWS
    iota = lax.iota(jnp.int32, 16)
    pat0 = iota * VOCAB
    lane1 = iota + 16
    cat1 = lane1 < N_CAT
    pat1 = jnp.where(cat1, lane1 * VOCAB, 0)

    a_in0 = pltpu.async_copy(inp_hbm.at[pl.ds(row0, ROWS_H)], inp_v, sem2)

    def ct_idx(j, carry):
        e = jnp.minimum(sid * CT_PER + j * 16 + iota, CTMAX)
        ctidx_v[pl.ds(j * 16, 16)] = e * EMB
        return carry
    lax.fori_loop(0, CT_VEC, ct_idx, None)
    a_ctab = pltpu.async_copy(tbl_hbm.at[ctidx_v], ctg_v, sem)

    def idx_loop(r, carry):
        v0 = inp_v[r, pl.ds(0, 16)]
        v1 = inp_v[r, pl.ds(16, 16)]
        idx_v[pl.ds(r * 32, 16)] = v0 + pat0
        idx_v[pl.ds(r * 32 + 16, 16)] = v1 + pat1
        return carry

    def merge_loop(r, carry):
        out_v[r, pl.ds(0, 16)] = gath_v[pl.ds(r * 32, 16)]
        v1 = inp_v[r, pl.ds(16, 16)].astype(jnp.float32)
        g1 = gath_v[pl.ds(r * 32 + 16, 16)]
        out_v[r, pl.ds(16, 16)] = jnp.where(cat1, g1, v1)
        out_v[r, pl.ds(32, 16)] = inp_v[r, pl.ds(32, 16)].astype(jnp.float32)
        return carry

    a_in0.wait()
    lax.fori_loop(0, ROWS_H, idx_loop, None)

    a_ctab.wait()
    pltpu.sync_copy(ctg_v, ctab_s.at[pl.ds(sid * CT_PER, CT_PER)])
    plsc.subcore_barrier()

    a_g0 = pltpu.async_copy(ctab_s.at[idx_v], gath_v, sem)
    a_g0.wait()
    lax.fori_loop(0, ROWS_H, merge_loop, None)
    a_out0 = pltpu.async_copy(out_v, out_hbm.at[pl.ds(row0, ROWS_H)], sem2)
    a_out0.wait()

    pltpu.sync_copy(inp_hbm.at[pl.ds(row0 + ROWS_H, ROWS_H)], inp_v)
    lax.fori_loop(0, ROWS_H, idx_loop, None)
    pltpu.async_copy(ctab_s.at[idx_v], gath_v, sem).wait()
    lax.fori_loop(0, ROWS_H, merge_loop, None)
    pltpu.sync_copy(out_v, out_hbm.at[pl.ds(row0 + ROWS_H, ROWS_H)])


def kernel(inputs, tables):
    mesh = plsc.VectorSubcoreMesh(core_axis_name="c", subcore_axis_name="s")
    k = pl.kernel(
        _body,
        mesh=mesh,
        out_type=jax.ShapeDtypeStruct((B, LANES), jnp.float32),
        scratch_types=[
            pltpu.VMEM((ROWS_H, LANES), jnp.int32),
            pltpu.VMEM((ROWS_H, LANES), jnp.float32),
            pltpu.VMEM((IDX_H,), jnp.int32),
            pltpu.VMEM((IDX_H,), jnp.float32),
            pltpu.VMEM((CT_PER,), jnp.int32),
            pltpu.VMEM((CT_PER,), jnp.float32),
            pltpu.VMEM_SHARED((NS * CT_PER,), jnp.float32),
            pltpu.SemaphoreType.DMA,
            pltpu.SemaphoreType.DMA,
        ],
    )
    inp_pad = jnp.pad(inputs, ((0, 0), (0, LANES - N_COLS)))
    out_pad = k(inp_pad, tables.reshape(-1))
    return out_pad[:, :N_COLS]
